# baseline (speedup 1.0000x reference)
import functools

import jax
import jax.numpy as jnp
from jax import lax
from jax.experimental import pallas as pl
from jax.experimental.pallas import tpu as pltpu
from jax.experimental.pallas import tpu_sc as plsc

_N = 10000
_E = 160000
_D = 256
_H = 512
_G = 64
_EPS = 1e-7

_EB = 80
_EPT = _E // 16
_NB = _EPT // _EB
_RPT = _N // 16
_RC = 16
_BN = 2000


def _sc_msgpass(xq, srcdst3, tvec, zrows):
    mesh = plsc.VectorSubcoreMesh(core_axis_name="c", subcore_axis_name="s")

    @functools.partial(
        pl.kernel,
        out_type=jax.ShapeDtypeStruct((4, _N, 64), jnp.float32),
        mesh=mesh,
        compiler_params=pltpu.CompilerParams(use_tc_tiling_on_sc=False),
        scratch_types=[
            pltpu.VMEM_SHARED((_N, 128), jnp.float32),
            pltpu.VMEM((_NB, _EB), jnp.int32),
            pltpu.VMEM((_EB,), jnp.int32),
            pltpu.VMEM((_EB,), jnp.int32),
            pltpu.VMEM((_EB,), jnp.int32),
            pltpu.VMEM((_EB,), jnp.int32),
            pltpu.VMEM((_EB, 64), jnp.float32),
            pltpu.VMEM((_EB, 64), jnp.float32),
            pltpu.VMEM((_EB, 128), jnp.float32),
            pltpu.VMEM((_EB, 128), jnp.float32),
            pltpu.VMEM((_RC, 128), jnp.float32),
            pltpu.VMEM((_RC, 64), jnp.float32),
            pltpu.VMEM((16,), jnp.float32),
            pltpu.SemaphoreType.DMA,
            pltpu.SemaphoreType.DMA,
            pltpu.SemaphoreType.DMA,
            pltpu.SemaphoreType.DMA,
        ],
    )
    def k(xq_hbm, sd_hbm, t_hbm, z_hbm, aggr_hbm,
          acc, sdall, gidx0, gidx1, dstb0, dstb1, xb0, xb1, ob0, ob1,
          rb, outb, tv, gsem0, gsem1, ssem0, ssem1):
        c = lax.axis_index("c")
        s = lax.axis_index("s")
        pltpu.sync_copy(t_hbm, tv)
        tq = tv[...]
        pltpu.sync_copy(sd_hbm.at[s], sdall)
        rbase = s * _RPT
        slots = ((gidx0, xb0, gsem0, ob0, ssem0, dstb0),
                 (gidx1, xb1, gsem1, ob1, ssem1, dstb1))

        for p in range(2):
            q = c * 2 + p
            qoff = q * _N
            pltpu.sync_copy(z_hbm, acc.at[pl.ds(rbase, _RPT)])
            plsc.subcore_barrier()

            def fill_gidx(gid, i):
                for v in range(_EB // 16):
                    gid[pl.ds(v * 16, 16)] = (
                        (sdall[i, pl.ds(v * 16, 16)] & 0xFFFF) + qoff)

            def fill_dst(db, i):
                for v in range(_EB // 16):
                    db[pl.ds(v * 16, 16)] = lax.shift_right_logical(
                        sdall[i, pl.ds(v * 16, 16)], 16)

            def process(b, i):
                gid, xb, gsem, ob, ssem, db = slots[b]
                pltpu.make_async_copy(xq_hbm.at[gid], xb, gsem).wait()

                @pl.when(i >= 2)
                def _():
                    pltpu.make_async_copy(ob, acc.at[db], ssem).wait()

                fill_dst(db, i)

                @plsc.parallel_loop(0, _EB, unroll=8)
                def edge_body(e):
                    for j in range(4):
                        v = xb[e, pl.ds(j * 16, 16)]
                        ex = jnp.exp(v)
                        ob[e, pl.ds(j * 16, 16)] = v * ex
                        ob[e, pl.ds(64 + j * 16, 16)] = ex
                @pl.when(i + 2 < _NB)
                def _():
                    fill_gidx(gid, i + 2)
                    pltpu.async_copy(xq_hbm.at[gid], xb, gsem)

                pltpu.async_copy(ob, acc.at[db], ssem, add=True)

            fill_gidx(gidx0, 0)
            pltpu.async_copy(xq_hbm.at[gidx0], xb0, gsem0)
            fill_gidx(gidx1, 1)
            pltpu.async_copy(xq_hbm.at[gidx1], xb1, gsem1)

            def batch_body(io, carry):
                for b in range(2):
                    process(b, io * 2 + b)
                return carry

            lax.fori_loop(0, _NB // 2, batch_body, 0)
            process(0, _NB - 1)
            pltpu.make_async_copy(ob0, acc.at[dstb0], ssem0).wait()
            pltpu.make_async_copy(ob1, acc.at[dstb1], ssem1).wait()
            plsc.subcore_barrier()

            def emit_chunk(r0, nrows):
                pltpu.sync_copy(acc.at[pl.ds(r0, nrows)],
                                rb.at[pl.ds(0, nrows)])

                @plsc.parallel_loop(0, nrows, unroll=4)
                def row_body(r):
                    for j in range(4):
                        num = rb[r, pl.ds(j * 16, 16)]
                        den = rb[r, pl.ds(64 + j * 16, 16)]
                        outb[r, pl.ds(j * 16, 16)] = num / (den * tq + 1e-16)
                pltpu.sync_copy(outb.at[pl.ds(0, nrows)],
                                aggr_hbm.at[q, pl.ds(r0, nrows)])

            def chunk_body(ch, carry4):
                emit_chunk(pl.multiple_of(s * 624 + ch * _RC, _RC), _RC)
                return carry4

            lax.fori_loop(0, 39, chunk_body, 0)

            @pl.when(s == 15)
            def _():
                emit_chunk(15 * 624 + 39 * _RC, 16)

            plsc.subcore_barrier()

    return k(xq, srcdst3, tvec, zrows)


def _tc_mlp1(aggr, x, W1, b1):
    def kern(aggr_ref, x_ref, w_ref, b_ref, h_ref, st_ref):
        i = pl.program_id(0)
        a4 = aggr_ref[...]
        aggr_blk = jnp.concatenate([a4[j] for j in range(4)], axis=-1)
        out = aggr_blk + x_ref[...]
        h = jnp.dot(out, w_ref[...], preferred_element_type=jnp.float32)
        h = h + b_ref[...]
        h_ref[...] = h
        ssum = jnp.sum(h, axis=0, keepdims=True)
        ssq = jnp.sum(h * h, axis=0, keepdims=True)
        st = jnp.concatenate(
            [ssum, ssq, jnp.zeros((6, _H), jnp.float32)], axis=0)

        @pl.when(i == 0)
        def _():
            st_ref[...] = st

        @pl.when(i > 0)
        def _():
            st_ref[...] += st

    return pl.pallas_call(
        kern,
        grid=(_N // _BN,),
        in_specs=[
            pl.BlockSpec((4, _BN, 64), lambda i: (0, i, 0)),
            pl.BlockSpec((_BN, _D), lambda i: (i, 0)),
            pl.BlockSpec((_D, _H), lambda i: (0, 0)),
            pl.BlockSpec((1, _H), lambda i: (0, 0)),
        ],
        out_specs=[
            pl.BlockSpec((_BN, _H), lambda i: (i, 0)),
            pl.BlockSpec((8, _H), lambda i: (0, 0)),
        ],
        out_shape=[
            jax.ShapeDtypeStruct((_N, _H), jnp.float32),
            jax.ShapeDtypeStruct((8, _H), jnp.float32),
        ],
    )(aggr, x, W1, b1)


def _tc_mlp2(h1, stats, gamma, beta, W2, b2, x, tnext):
    def kern(h_ref, st_ref, g_ref, be_ref, w_ref, b_ref, x_ref, t_ref,
             xn_ref, xq_ref):
        mu = st_ref[0:1, :] * (1.0 / _N)
        var = st_ref[1:2, :] * (1.0 / _N) - mu * mu
        hn = (h_ref[...] - mu) / jnp.sqrt(var + 1e-5) * g_ref[...] + be_ref[...]
        hr = jnp.maximum(hn, 0.0)
        h2 = jnp.dot(hr, w_ref[...], preferred_element_type=jnp.float32)
        xn = x_ref[...] + h2 + b_ref[...]
        xn_ref[...] = xn
        v = (jnp.maximum(xn, 0.0) + _EPS) * t_ref[0, 0]
        for j in range(4):
            xq_ref[j, :, :] = v[:, j * 64:(j + 1) * 64]

    return pl.pallas_call(
        kern,
        grid=(_N // _BN,),
        in_specs=[
            pl.BlockSpec((_BN, _H), lambda i: (i, 0)),
            pl.BlockSpec((8, _H), lambda i: (0, 0)),
            pl.BlockSpec((1, _H), lambda i: (0, 0)),
            pl.BlockSpec((1, _H), lambda i: (0, 0)),
            pl.BlockSpec((_H, _D), lambda i: (0, 0)),
            pl.BlockSpec((1, _D), lambda i: (0, 0)),
            pl.BlockSpec((_BN, _D), lambda i: (i, 0)),
            pl.BlockSpec((1, 1), lambda i: (0, 0)),
        ],
        out_specs=[
            pl.BlockSpec((_BN, _D), lambda i: (i, 0)),
            pl.BlockSpec((4, _BN, 64), lambda i: (0, i, 0)),
        ],
        out_shape=[
            jax.ShapeDtypeStruct((_N, _D), jnp.float32),
            jax.ShapeDtypeStruct((4, _N, 64), jnp.float32),
        ],
    )(h1, stats, gamma, beta, W2, b2, x, tnext)


def _tc_prep(x, t):
    def kern(x_ref, t_ref, xq_ref):
        v = (jnp.maximum(x_ref[...], 0.0) + _EPS) * t_ref[0, 0]
        for j in range(4):
            xq_ref[j, :, :] = v[:, j * 64:(j + 1) * 64]

    return pl.pallas_call(
        kern,
        grid=(_N // _BN,),
        in_specs=[
            pl.BlockSpec((_BN, _D), lambda i: (i, 0)),
            pl.BlockSpec((1, 1), lambda i: (0, 0)),
        ],
        out_specs=pl.BlockSpec((4, _BN, 64), lambda i: (0, i, 0)),
        out_shape=jax.ShapeDtypeStruct((4, _N, 64), jnp.float32),
    )(x, t)


def _tc_pool(h, batch2d, wp_row, bp):
    nblk = _N // _BN

    def kern(h_ref, b_ref, wp_ref, bp_ref, pool_ref, cnt_ref, out_ref):
        i = pl.program_id(0)
        bids = b_ref[...].reshape(_BN, 1)
        gids = lax.broadcasted_iota(jnp.int32, (_BN, _G), 1)
        oh = (bids == gids).astype(jnp.float32)
        contrib = lax.dot_general(oh, h_ref[...], (((0,), (0,)), ((), ())),
                                  preferred_element_type=jnp.float32)
        cnt = jnp.sum(oh, axis=0, keepdims=True)
        cnt8 = jnp.concatenate([cnt, jnp.zeros((7, _G), jnp.float32)], axis=0)

        @pl.when(i == 0)
        def _():
            pool_ref[...] = contrib
            cnt_ref[...] = cnt8

        @pl.when(i > 0)
        def _():
            pool_ref[...] += contrib
            cnt_ref[...] += cnt8

        @pl.when(i == nblk - 1)
        def _():
            cnt_f = jnp.maximum(cnt_ref[0:1, :], 1.0)
            pooled = pool_ref[...] / cnt_f.reshape(_G, 1)
            o = jnp.sum(pooled * wp_ref[...], axis=1, keepdims=True)
            o = o + bp_ref[0:1, 0:1]
            out_ref[...] = jnp.broadcast_to(o, (_G, 128))

    pool, cnt, out = pl.pallas_call(
        kern,
        grid=(nblk,),
        in_specs=[
            pl.BlockSpec((_BN, _D), lambda i: (i, 0)),
            pl.BlockSpec((1, 1, _BN), lambda i: (i, 0, 0)),
            pl.BlockSpec((1, _D), lambda i: (0, 0)),
            pl.BlockSpec((1, 1), lambda i: (0, 0)),
        ],
        out_specs=[
            pl.BlockSpec((_G, _D), lambda i: (0, 0)),
            pl.BlockSpec((8, _G), lambda i: (0, 0)),
            pl.BlockSpec((_G, 128), lambda i: (0, 0)),
        ],
        out_shape=[
            jax.ShapeDtypeStruct((_G, _D), jnp.float32),
            jax.ShapeDtypeStruct((8, _G), jnp.float32),
            jax.ShapeDtypeStruct((_G, 128), jnp.float32),
        ],
    )(h, batch2d, wp_row, bp)
    return out[:, 0:1]


def kernel(x, edge_index, batch, t0, W1_0, b1_0, gamma0, beta0, W2_0, b2_0,
           t1, W1_1, b1_1, gamma1, beta1, W2_1, b2_1,
           t2, W1_2, b1_2, gamma2, beta2, W2_2, b2_2, Wp, bp):
    srcdst3 = (edge_index[0] | (edge_index[1] << 16)).reshape(16, _NB, _EB)
    zrows = jnp.zeros((_RPT, 128), jnp.float32)
    h = x
    ts = [t0, t1, t2, jnp.array(1.0, jnp.float32)]
    xq = _tc_prep(x, ts[0].reshape(1, 1)).reshape(4 * _N, 64)
    layers = [
        (t0, W1_0, b1_0, gamma0, beta0, W2_0, b2_0),
        (t1, W1_1, b1_1, gamma1, beta1, W2_1, b2_1),
        (t2, W1_2, b1_2, gamma2, beta2, W2_2, b2_2),
    ]
    for li, (t, W1, b1, gamma, beta, W2, b2) in enumerate(layers):
        tvec = jnp.broadcast_to(t, (16,)).astype(jnp.float32)
        aggr = _sc_msgpass(xq, srcdst3, tvec, zrows)
        h1, stats = _tc_mlp1(aggr, h, W1, b1.reshape(1, _H))
        h, xq4 = _tc_mlp2(h1, stats, gamma.reshape(1, _H), beta.reshape(1, _H),
                          W2, b2.reshape(1, _D), h,
                          ts[li + 1].reshape(1, 1))
        xq = xq4.reshape(4 * _N, 64)
    return _tc_pool(h, batch.reshape(_N // _BN, 1, _BN), Wp.reshape(1, _D),
                    bp.reshape(1, 1))


# speedup vs baseline: 1.0460x; 1.0460x over previous
import functools

import jax
import jax.numpy as jnp
from jax import lax
from jax.experimental import pallas as pl
from jax.experimental.pallas import tpu as pltpu
from jax.experimental.pallas import tpu_sc as plsc

_N = 10000
_E = 160000
_D = 256
_H = 512
_G = 64
_EPS = 1e-7

_EB = 80
_EPT = _E // 16
_NB = _EPT // _EB
_RPT = _N // 16
_RC = 16
_BN = 2000


def _sc_msgpass(xq, srcdst3, tvec, zrows):
    mesh = plsc.VectorSubcoreMesh(core_axis_name="c", subcore_axis_name="s")

    @functools.partial(
        pl.kernel,
        out_type=jax.ShapeDtypeStruct((4, _N, 64), jnp.float32),
        mesh=mesh,
        compiler_params=pltpu.CompilerParams(use_tc_tiling_on_sc=False),
        scratch_types=[
            pltpu.VMEM_SHARED((_N, 128), jnp.float32),
            pltpu.VMEM((_NB, _EB), jnp.int32),
            pltpu.VMEM((_EB,), jnp.int32),
            pltpu.VMEM((_EB,), jnp.int32),
            pltpu.VMEM((_EB,), jnp.int32),
            pltpu.VMEM((_EB,), jnp.int32),
            pltpu.VMEM((_EB, 64), jnp.float32),
            pltpu.VMEM((_EB, 64), jnp.float32),
            pltpu.VMEM((_EB, 128), jnp.float32),
            pltpu.VMEM((_EB, 128), jnp.float32),
            pltpu.VMEM((_RC, 128), jnp.float32),
            pltpu.VMEM((_RC, 64), jnp.float32),
            pltpu.VMEM((16,), jnp.float32),
            pltpu.SemaphoreType.DMA,
            pltpu.SemaphoreType.DMA,
            pltpu.SemaphoreType.DMA,
            pltpu.SemaphoreType.DMA,
        ],
    )
    def k(xq_hbm, sd_hbm, t_hbm, z_hbm, aggr_hbm,
          acc, sdall, gidx0, gidx1, dstb0, dstb1, xb0, xb1, ob0, ob1,
          rb, outb, tv, gsem0, gsem1, ssem0, ssem1):
        c = lax.axis_index("c")
        s = lax.axis_index("s")
        pltpu.sync_copy(t_hbm, tv)
        tq = tv[...]
        pltpu.sync_copy(sd_hbm.at[s], sdall)
        rbase = s * _RPT
        slots = ((gidx0, xb0, gsem0, ob0, ssem0, dstb0),
                 (gidx1, xb1, gsem1, ob1, ssem1, dstb1))

        for p in range(2):
            q = c * 2 + p
            qoff = q * _N
            pltpu.sync_copy(z_hbm, acc.at[pl.ds(rbase, _RPT)])
            plsc.subcore_barrier()

            def fill_gidx(gid, i):
                for v in range(_EB // 16):
                    gid[pl.ds(v * 16, 16)] = (
                        (sdall[i, pl.ds(v * 16, 16)] & 0xFFFF) + qoff)

            def fill_dst(db, i):
                for v in range(_EB // 16):
                    db[pl.ds(v * 16, 16)] = lax.shift_right_logical(
                        sdall[i, pl.ds(v * 16, 16)], 16)

            def process(b, i):
                gid, xb, gsem, ob, ssem, db = slots[b]
                pltpu.make_async_copy(xq_hbm.at[gid], xb, gsem).wait()

                @pl.when(i >= 2)
                def _():
                    pltpu.make_async_copy(ob, acc.at[db], ssem).wait()

                fill_dst(db, i)

                @plsc.parallel_loop(0, _EB, unroll=8)
                def edge_body(e):
                    for j in range(4):
                        v = xb[e, pl.ds(j * 16, 16)]
                        ex = jnp.exp(v)
                        ob[e, pl.ds(j * 16, 16)] = v * ex
                        ob[e, pl.ds(64 + j * 16, 16)] = ex
                @pl.when(i + 2 < _NB)
                def _():
                    fill_gidx(gid, i + 2)
                    pltpu.async_copy(xq_hbm.at[gid], xb, gsem)

                pltpu.async_copy(ob, acc.at[db], ssem, add=True)

            fill_gidx(gidx0, 0)
            pltpu.async_copy(xq_hbm.at[gidx0], xb0, gsem0)
            fill_gidx(gidx1, 1)
            pltpu.async_copy(xq_hbm.at[gidx1], xb1, gsem1)

            def batch_body(io, carry):
                for b in range(2):
                    process(b, io * 2 + b)
                return carry

            lax.fori_loop(0, _NB // 2, batch_body, 0)
            process(0, _NB - 1)
            pltpu.make_async_copy(ob0, acc.at[dstb0], ssem0).wait()
            pltpu.make_async_copy(ob1, acc.at[dstb1], ssem1).wait()
            plsc.subcore_barrier()

            eslots = ((ob0, gsem0, xb0, ssem0), (ob1, gsem1, xb1, ssem1))

            def rd_chunk(i, rbuf, rsem):
                pltpu.async_copy(
                    acc.at[pl.ds(pl.multiple_of(s * 624 + i * _RC, _RC), _RC)],
                    rbuf.at[pl.ds(0, _RC)], rsem)

            def process_chunk(b, i):
                rbuf, rsem, obuf, wsem = eslots[b]
                pltpu.make_async_copy(
                    acc.at[pl.ds(pl.multiple_of(s * 624 + i * _RC, _RC), _RC)],
                    rbuf.at[pl.ds(0, _RC)], rsem).wait()

                @pl.when(i >= 2)
                def _():
                    pltpu.make_async_copy(
                        obuf.at[pl.ds(0, _RC)],
                        aggr_hbm.at[q, pl.ds(
                            pl.multiple_of(s * 624 + (i - 2) * _RC, _RC),
                            _RC)],
                        wsem).wait()

                @plsc.parallel_loop(0, _RC, unroll=4)
                def row_body(r):
                    for j in range(4):
                        num = rbuf[r, pl.ds(j * 16, 16)]
                        den = rbuf[r, pl.ds(64 + j * 16, 16)]
                        obuf[r, pl.ds(j * 16, 16)] = num / (den * tq + 1e-16)

                pltpu.async_copy(
                    obuf.at[pl.ds(0, _RC)],
                    aggr_hbm.at[q, pl.ds(
                        pl.multiple_of(s * 624 + i * _RC, _RC), _RC)], wsem)

                @pl.when(i + 2 < 39)
                def _():
                    rd_chunk(i + 2, rbuf, rsem)

            rd_chunk(0, ob0, gsem0)
            rd_chunk(1, ob1, gsem1)

            def chunk_body(io, carry4):
                for b in range(2):
                    process_chunk(b, io * 2 + b)
                return carry4

            lax.fori_loop(0, 19, chunk_body, 0)
            process_chunk(0, 38)
            pltpu.make_async_copy(
                xb0.at[pl.ds(0, _RC)],
                aggr_hbm.at[q, pl.ds(
                    pl.multiple_of(s * 624 + 38 * _RC, _RC), _RC)],
                ssem0).wait()
            pltpu.make_async_copy(
                xb1.at[pl.ds(0, _RC)],
                aggr_hbm.at[q, pl.ds(
                    pl.multiple_of(s * 624 + 37 * _RC, _RC), _RC)],
                ssem1).wait()

            @pl.when(s == 15)
            def _():
                r0 = 15 * 624 + 39 * _RC
                pltpu.sync_copy(acc.at[pl.ds(r0, 16)], rb.at[pl.ds(0, 16)])

                @plsc.parallel_loop(0, 16, unroll=4)
                def row_body(r):
                    for j in range(4):
                        num = rb[r, pl.ds(j * 16, 16)]
                        den = rb[r, pl.ds(64 + j * 16, 16)]
                        outb[r, pl.ds(j * 16, 16)] = num / (den * tq + 1e-16)
                pltpu.sync_copy(outb.at[pl.ds(0, 16)],
                                aggr_hbm.at[q, pl.ds(r0, 16)])

            plsc.subcore_barrier()

    return k(xq, srcdst3, tvec, zrows)


def _tc_mlp1(aggr, x, W1, b1):
    def kern(aggr_ref, x_ref, w_ref, b_ref, h_ref, st_ref):
        i = pl.program_id(0)
        a4 = aggr_ref[...]
        aggr_blk = jnp.concatenate([a4[j] for j in range(4)], axis=-1)
        out = aggr_blk + x_ref[...]
        h = jnp.dot(out, w_ref[...], preferred_element_type=jnp.float32)
        h = h + b_ref[...]
        h_ref[...] = h
        ssum = jnp.sum(h, axis=0, keepdims=True)
        ssq = jnp.sum(h * h, axis=0, keepdims=True)
        st = jnp.concatenate(
            [ssum, ssq, jnp.zeros((6, _H), jnp.float32)], axis=0)

        @pl.when(i == 0)
        def _():
            st_ref[...] = st

        @pl.when(i > 0)
        def _():
            st_ref[...] += st

    return pl.pallas_call(
        kern,
        grid=(_N // _BN,),
        in_specs=[
            pl.BlockSpec((4, _BN, 64), lambda i: (0, i, 0)),
            pl.BlockSpec((_BN, _D), lambda i: (i, 0)),
            pl.BlockSpec((_D, _H), lambda i: (0, 0)),
            pl.BlockSpec((1, _H), lambda i: (0, 0)),
        ],
        out_specs=[
            pl.BlockSpec((_BN, _H), lambda i: (i, 0)),
            pl.BlockSpec((8, _H), lambda i: (0, 0)),
        ],
        out_shape=[
            jax.ShapeDtypeStruct((_N, _H), jnp.float32),
            jax.ShapeDtypeStruct((8, _H), jnp.float32),
        ],
    )(aggr, x, W1, b1)


def _tc_mlp2(h1, stats, gamma, beta, W2, b2, x, tnext):
    def kern(h_ref, st_ref, g_ref, be_ref, w_ref, b_ref, x_ref, t_ref,
             xn_ref, xq_ref):
        mu = st_ref[0:1, :] * (1.0 / _N)
        var = st_ref[1:2, :] * (1.0 / _N) - mu * mu
        hn = (h_ref[...] - mu) / jnp.sqrt(var + 1e-5) * g_ref[...] + be_ref[...]
        hr = jnp.maximum(hn, 0.0)
        h2 = jnp.dot(hr, w_ref[...], preferred_element_type=jnp.float32)
        xn = x_ref[...] + h2 + b_ref[...]
        xn_ref[...] = xn
        v = (jnp.maximum(xn, 0.0) + _EPS) * t_ref[0, 0]
        for j in range(4):
            xq_ref[j, :, :] = v[:, j * 64:(j + 1) * 64]

    return pl.pallas_call(
        kern,
        grid=(_N // _BN,),
        in_specs=[
            pl.BlockSpec((_BN, _H), lambda i: (i, 0)),
            pl.BlockSpec((8, _H), lambda i: (0, 0)),
            pl.BlockSpec((1, _H), lambda i: (0, 0)),
            pl.BlockSpec((1, _H), lambda i: (0, 0)),
            pl.BlockSpec((_H, _D), lambda i: (0, 0)),
            pl.BlockSpec((1, _D), lambda i: (0, 0)),
            pl.BlockSpec((_BN, _D), lambda i: (i, 0)),
            pl.BlockSpec((1, 1), lambda i: (0, 0)),
        ],
        out_specs=[
            pl.BlockSpec((_BN, _D), lambda i: (i, 0)),
            pl.BlockSpec((4, _BN, 64), lambda i: (0, i, 0)),
        ],
        out_shape=[
            jax.ShapeDtypeStruct((_N, _D), jnp.float32),
            jax.ShapeDtypeStruct((4, _N, 64), jnp.float32),
        ],
    )(h1, stats, gamma, beta, W2, b2, x, tnext)


def _tc_prep(x, t):
    def kern(x_ref, t_ref, xq_ref):
        v = (jnp.maximum(x_ref[...], 0.0) + _EPS) * t_ref[0, 0]
        for j in range(4):
            xq_ref[j, :, :] = v[:, j * 64:(j + 1) * 64]

    return pl.pallas_call(
        kern,
        grid=(_N // _BN,),
        in_specs=[
            pl.BlockSpec((_BN, _D), lambda i: (i, 0)),
            pl.BlockSpec((1, 1), lambda i: (0, 0)),
        ],
        out_specs=pl.BlockSpec((4, _BN, 64), lambda i: (0, i, 0)),
        out_shape=jax.ShapeDtypeStruct((4, _N, 64), jnp.float32),
    )(x, t)


def _tc_pool(h, batch2d, wp_row, bp):
    nblk = _N // _BN

    def kern(h_ref, b_ref, wp_ref, bp_ref, pool_ref, cnt_ref, out_ref):
        i = pl.program_id(0)
        bids = b_ref[...].reshape(_BN, 1)
        gids = lax.broadcasted_iota(jnp.int32, (_BN, _G), 1)
        oh = (bids == gids).astype(jnp.float32)
        contrib = lax.dot_general(oh, h_ref[...], (((0,), (0,)), ((), ())),
                                  preferred_element_type=jnp.float32)
        cnt = jnp.sum(oh, axis=0, keepdims=True)
        cnt8 = jnp.concatenate([cnt, jnp.zeros((7, _G), jnp.float32)], axis=0)

        @pl.when(i == 0)
        def _():
            pool_ref[...] = contrib
            cnt_ref[...] = cnt8

        @pl.when(i > 0)
        def _():
            pool_ref[...] += contrib
            cnt_ref[...] += cnt8

        @pl.when(i == nblk - 1)
        def _():
            cnt_f = jnp.maximum(cnt_ref[0:1, :], 1.0)
            pooled = pool_ref[...] / cnt_f.reshape(_G, 1)
            o = jnp.sum(pooled * wp_ref[...], axis=1, keepdims=True)
            o = o + bp_ref[0:1, 0:1]
            out_ref[...] = jnp.broadcast_to(o, (_G, 128))

    pool, cnt, out = pl.pallas_call(
        kern,
        grid=(nblk,),
        in_specs=[
            pl.BlockSpec((_BN, _D), lambda i: (i, 0)),
            pl.BlockSpec((1, 1, _BN), lambda i: (i, 0, 0)),
            pl.BlockSpec((1, _D), lambda i: (0, 0)),
            pl.BlockSpec((1, 1), lambda i: (0, 0)),
        ],
        out_specs=[
            pl.BlockSpec((_G, _D), lambda i: (0, 0)),
            pl.BlockSpec((8, _G), lambda i: (0, 0)),
            pl.BlockSpec((_G, 128), lambda i: (0, 0)),
        ],
        out_shape=[
            jax.ShapeDtypeStruct((_G, _D), jnp.float32),
            jax.ShapeDtypeStruct((8, _G), jnp.float32),
            jax.ShapeDtypeStruct((_G, 128), jnp.float32),
        ],
    )(h, batch2d, wp_row, bp)
    return out[:, 0:1]


def kernel(x, edge_index, batch, t0, W1_0, b1_0, gamma0, beta0, W2_0, b2_0,
           t1, W1_1, b1_1, gamma1, beta1, W2_1, b2_1,
           t2, W1_2, b1_2, gamma2, beta2, W2_2, b2_2, Wp, bp):
    srcdst3 = (edge_index[0] | (edge_index[1] << 16)).reshape(16, _NB, _EB)
    zrows = jnp.zeros((_RPT, 128), jnp.float32)
    h = x
    ts = [t0, t1, t2, jnp.array(1.0, jnp.float32)]
    xq = _tc_prep(x, ts[0].reshape(1, 1)).reshape(4 * _N, 64)
    layers = [
        (t0, W1_0, b1_0, gamma0, beta0, W2_0, b2_0),
        (t1, W1_1, b1_1, gamma1, beta1, W2_1, b2_1),
        (t2, W1_2, b1_2, gamma2, beta2, W2_2, b2_2),
    ]
    for li, (t, W1, b1, gamma, beta, W2, b2) in enumerate(layers):
        tvec = jnp.broadcast_to(t, (16,)).astype(jnp.float32)
        aggr = _sc_msgpass(xq, srcdst3, tvec, zrows)
        h1, stats = _tc_mlp1(aggr, h, W1, b1.reshape(1, _H))
        h, xq4 = _tc_mlp2(h1, stats, gamma.reshape(1, _H), beta.reshape(1, _H),
                          W2, b2.reshape(1, _D), h,
                          ts[li + 1].reshape(1, 1))
        xq = xq4.reshape(4 * _N, 64)
    return _tc_pool(h, batch.reshape(_N // _BN, 1, _BN), Wp.reshape(1, _D),
                    bp.reshape(1, 1))


# speedup vs baseline: 1.0516x; 1.0053x over previous
import functools

import jax
import jax.numpy as jnp
from jax import lax
from jax.experimental import pallas as pl
from jax.experimental.pallas import tpu as pltpu
from jax.experimental.pallas import tpu_sc as plsc

_N = 10000
_E = 160000
_D = 256
_H = 512
_G = 64
_EPS = 1e-7

_EB = 80
_EPT = _E // 16
_NB = _EPT // _EB
_RPT = _N // 16
_RC = 48
_BN = 2000


def _sc_msgpass(xq, srcdst3, tvec, zrows):
    mesh = plsc.VectorSubcoreMesh(core_axis_name="c", subcore_axis_name="s")

    @functools.partial(
        pl.kernel,
        out_type=jax.ShapeDtypeStruct((4, _N, 64), jnp.float32),
        mesh=mesh,
        compiler_params=pltpu.CompilerParams(use_tc_tiling_on_sc=False),
        scratch_types=[
            pltpu.VMEM_SHARED((_N, 128), jnp.float32),
            pltpu.VMEM((_NB, _EB), jnp.int32),
            pltpu.VMEM((_EB,), jnp.int32),
            pltpu.VMEM((_EB,), jnp.int32),
            pltpu.VMEM((_EB,), jnp.int32),
            pltpu.VMEM((_EB,), jnp.int32),
            pltpu.VMEM((_EB, 64), jnp.float32),
            pltpu.VMEM((_EB, 64), jnp.float32),
            pltpu.VMEM((_EB, 128), jnp.float32),
            pltpu.VMEM((_EB, 128), jnp.float32),
            pltpu.VMEM((16, 128), jnp.float32),
            pltpu.VMEM((16, 64), jnp.float32),
            pltpu.VMEM((16,), jnp.float32),
            pltpu.SemaphoreType.DMA,
            pltpu.SemaphoreType.DMA,
            pltpu.SemaphoreType.DMA,
            pltpu.SemaphoreType.DMA,
        ],
    )
    def k(xq_hbm, sd_hbm, t_hbm, z_hbm, aggr_hbm,
          acc, sdall, gidx0, gidx1, dstb0, dstb1, xb0, xb1, ob0, ob1,
          rb, outb, tv, gsem0, gsem1, ssem0, ssem1):
        c = lax.axis_index("c")
        s = lax.axis_index("s")
        pltpu.sync_copy(t_hbm, tv)
        tq = tv[...]
        pltpu.sync_copy(sd_hbm.at[s], sdall)
        rbase = s * _RPT
        slots = ((gidx0, xb0, gsem0, ob0, ssem0, dstb0),
                 (gidx1, xb1, gsem1, ob1, ssem1, dstb1))

        for p in range(2):
            q = c * 2 + p
            qoff = q * _N
            pltpu.sync_copy(z_hbm, acc.at[pl.ds(rbase, _RPT)])
            plsc.subcore_barrier()

            def fill_gidx(gid, i):
                for v in range(_EB // 16):
                    gid[pl.ds(v * 16, 16)] = (
                        (sdall[i, pl.ds(v * 16, 16)] & 0xFFFF) + qoff)

            def fill_dst(db, i):
                for v in range(_EB // 16):
                    db[pl.ds(v * 16, 16)] = lax.shift_right_logical(
                        sdall[i, pl.ds(v * 16, 16)], 16)

            def process(b, i):
                gid, xb, gsem, ob, ssem, db = slots[b]
                pltpu.make_async_copy(xq_hbm.at[gid], xb, gsem).wait()

                @pl.when(i >= 2)
                def _():
                    pltpu.make_async_copy(ob, acc.at[db], ssem).wait()

                fill_dst(db, i)

                @plsc.parallel_loop(0, _EB, unroll=8)
                def edge_body(e):
                    for j in range(4):
                        v = xb[e, pl.ds(j * 16, 16)]
                        ex = jnp.exp(v)
                        ob[e, pl.ds(j * 16, 16)] = v * ex
                        ob[e, pl.ds(64 + j * 16, 16)] = ex
                @pl.when(i + 2 < _NB)
                def _():
                    fill_gidx(gid, i + 2)
                    pltpu.async_copy(xq_hbm.at[gid], xb, gsem)

                pltpu.async_copy(ob, acc.at[db], ssem, add=True)

            fill_gidx(gidx0, 0)
            pltpu.async_copy(xq_hbm.at[gidx0], xb0, gsem0)
            fill_gidx(gidx1, 1)
            pltpu.async_copy(xq_hbm.at[gidx1], xb1, gsem1)

            def batch_body(io, carry):
                for b in range(2):
                    process(b, io * 2 + b)
                return carry

            lax.fori_loop(0, _NB // 2, batch_body, 0)
            process(0, _NB - 1)
            pltpu.make_async_copy(ob0, acc.at[dstb0], ssem0).wait()
            pltpu.make_async_copy(ob1, acc.at[dstb1], ssem1).wait()
            plsc.subcore_barrier()

            eslots = ((ob0, gsem0, xb0, ssem0), (ob1, gsem1, xb1, ssem1))

            def rd_chunk(i, rbuf, rsem):
                pltpu.async_copy(
                    acc.at[pl.ds(pl.multiple_of(s * 624 + i * _RC, _RC), _RC)],
                    rbuf.at[pl.ds(0, _RC)], rsem)

            def process_chunk(b, i):
                rbuf, rsem, obuf, wsem = eslots[b]
                pltpu.make_async_copy(
                    acc.at[pl.ds(pl.multiple_of(s * 624 + i * _RC, _RC), _RC)],
                    rbuf.at[pl.ds(0, _RC)], rsem).wait()

                @pl.when(i >= 2)
                def _():
                    pltpu.make_async_copy(
                        obuf.at[pl.ds(0, _RC)],
                        aggr_hbm.at[q, pl.ds(
                            pl.multiple_of(s * 624 + (i - 2) * _RC, _RC),
                            _RC)],
                        wsem).wait()

                @plsc.parallel_loop(0, _RC, unroll=4)
                def row_body(r):
                    for j in range(4):
                        num = rbuf[r, pl.ds(j * 16, 16)]
                        den = rbuf[r, pl.ds(64 + j * 16, 16)]
                        obuf[r, pl.ds(j * 16, 16)] = num / (den * tq + 1e-16)

                pltpu.async_copy(
                    obuf.at[pl.ds(0, _RC)],
                    aggr_hbm.at[q, pl.ds(
                        pl.multiple_of(s * 624 + i * _RC, _RC), _RC)], wsem)

                @pl.when(i + 2 < 13)
                def _():
                    rd_chunk(i + 2, rbuf, rsem)

            rd_chunk(0, ob0, gsem0)
            rd_chunk(1, ob1, gsem1)

            def chunk_body(io, carry4):
                for b in range(2):
                    process_chunk(b, io * 2 + b)
                return carry4

            lax.fori_loop(0, 6, chunk_body, 0)
            process_chunk(0, 12)
            pltpu.make_async_copy(
                xb0.at[pl.ds(0, _RC)],
                aggr_hbm.at[q, pl.ds(
                    pl.multiple_of(s * 624 + 12 * _RC, _RC), _RC)],
                ssem0).wait()
            pltpu.make_async_copy(
                xb1.at[pl.ds(0, _RC)],
                aggr_hbm.at[q, pl.ds(
                    pl.multiple_of(s * 624 + 11 * _RC, _RC), _RC)],
                ssem1).wait()

            @pl.when(s == 15)
            def _():
                r0 = 15 * 624 + 13 * _RC
                pltpu.sync_copy(acc.at[pl.ds(r0, 16)], rb.at[pl.ds(0, 16)])

                @plsc.parallel_loop(0, 16, unroll=4)
                def row_body(r):
                    for j in range(4):
                        num = rb[r, pl.ds(j * 16, 16)]
                        den = rb[r, pl.ds(64 + j * 16, 16)]
                        outb[r, pl.ds(j * 16, 16)] = num / (den * tq + 1e-16)
                pltpu.sync_copy(outb.at[pl.ds(0, 16)],
                                aggr_hbm.at[q, pl.ds(r0, 16)])

            plsc.subcore_barrier()

    return k(xq, srcdst3, tvec, zrows)


def _tc_mlp1(aggr, x, W1, b1):
    def kern(aggr_ref, x_ref, w_ref, b_ref, h_ref, st_ref):
        i = pl.program_id(0)
        a4 = aggr_ref[...]
        aggr_blk = jnp.concatenate([a4[j] for j in range(4)], axis=-1)
        out = aggr_blk + x_ref[...]
        h = jnp.dot(out, w_ref[...], preferred_element_type=jnp.float32)
        h = h + b_ref[...]
        h_ref[...] = h
        ssum = jnp.sum(h, axis=0, keepdims=True)
        ssq = jnp.sum(h * h, axis=0, keepdims=True)
        st = jnp.concatenate(
            [ssum, ssq, jnp.zeros((6, _H), jnp.float32)], axis=0)

        @pl.when(i == 0)
        def _():
            st_ref[...] = st

        @pl.when(i > 0)
        def _():
            st_ref[...] += st

    return pl.pallas_call(
        kern,
        grid=(_N // _BN,),
        in_specs=[
            pl.BlockSpec((4, _BN, 64), lambda i: (0, i, 0)),
            pl.BlockSpec((_BN, _D), lambda i: (i, 0)),
            pl.BlockSpec((_D, _H), lambda i: (0, 0)),
            pl.BlockSpec((1, _H), lambda i: (0, 0)),
        ],
        out_specs=[
            pl.BlockSpec((_BN, _H), lambda i: (i, 0)),
            pl.BlockSpec((8, _H), lambda i: (0, 0)),
        ],
        out_shape=[
            jax.ShapeDtypeStruct((_N, _H), jnp.float32),
            jax.ShapeDtypeStruct((8, _H), jnp.float32),
        ],
    )(aggr, x, W1, b1)


def _tc_mlp2(h1, stats, gamma, beta, W2, b2, x, tnext):
    def kern(h_ref, st_ref, g_ref, be_ref, w_ref, b_ref, x_ref, t_ref,
             xn_ref, xq_ref):
        mu = st_ref[0:1, :] * (1.0 / _N)
        var = st_ref[1:2, :] * (1.0 / _N) - mu * mu
        hn = (h_ref[...] - mu) / jnp.sqrt(var + 1e-5) * g_ref[...] + be_ref[...]
        hr = jnp.maximum(hn, 0.0)
        h2 = jnp.dot(hr, w_ref[...], preferred_element_type=jnp.float32)
        xn = x_ref[...] + h2 + b_ref[...]
        xn_ref[...] = xn
        v = (jnp.maximum(xn, 0.0) + _EPS) * t_ref[0, 0]
        for j in range(4):
            xq_ref[j, :, :] = v[:, j * 64:(j + 1) * 64]

    return pl.pallas_call(
        kern,
        grid=(_N // _BN,),
        in_specs=[
            pl.BlockSpec((_BN, _H), lambda i: (i, 0)),
            pl.BlockSpec((8, _H), lambda i: (0, 0)),
            pl.BlockSpec((1, _H), lambda i: (0, 0)),
            pl.BlockSpec((1, _H), lambda i: (0, 0)),
            pl.BlockSpec((_H, _D), lambda i: (0, 0)),
            pl.BlockSpec((1, _D), lambda i: (0, 0)),
            pl.BlockSpec((_BN, _D), lambda i: (i, 0)),
            pl.BlockSpec((1, 1), lambda i: (0, 0)),
        ],
        out_specs=[
            pl.BlockSpec((_BN, _D), lambda i: (i, 0)),
            pl.BlockSpec((4, _BN, 64), lambda i: (0, i, 0)),
        ],
        out_shape=[
            jax.ShapeDtypeStruct((_N, _D), jnp.float32),
            jax.ShapeDtypeStruct((4, _N, 64), jnp.float32),
        ],
    )(h1, stats, gamma, beta, W2, b2, x, tnext)


def _tc_prep(x, t):
    def kern(x_ref, t_ref, xq_ref):
        v = (jnp.maximum(x_ref[...], 0.0) + _EPS) * t_ref[0, 0]
        for j in range(4):
            xq_ref[j, :, :] = v[:, j * 64:(j + 1) * 64]

    return pl.pallas_call(
        kern,
        grid=(_N // _BN,),
        in_specs=[
            pl.BlockSpec((_BN, _D), lambda i: (i, 0)),
            pl.BlockSpec((1, 1), lambda i: (0, 0)),
        ],
        out_specs=pl.BlockSpec((4, _BN, 64), lambda i: (0, i, 0)),
        out_shape=jax.ShapeDtypeStruct((4, _N, 64), jnp.float32),
    )(x, t)


def _tc_pool(h, batch2d, wp_row, bp):
    nblk = _N // _BN

    def kern(h_ref, b_ref, wp_ref, bp_ref, pool_ref, cnt_ref, out_ref):
        i = pl.program_id(0)
        bids = b_ref[...].reshape(_BN, 1)
        gids = lax.broadcasted_iota(jnp.int32, (_BN, _G), 1)
        oh = (bids == gids).astype(jnp.float32)
        contrib = lax.dot_general(oh, h_ref[...], (((0,), (0,)), ((), ())),
                                  preferred_element_type=jnp.float32)
        cnt = jnp.sum(oh, axis=0, keepdims=True)
        cnt8 = jnp.concatenate([cnt, jnp.zeros((7, _G), jnp.float32)], axis=0)

        @pl.when(i == 0)
        def _():
            pool_ref[...] = contrib
            cnt_ref[...] = cnt8

        @pl.when(i > 0)
        def _():
            pool_ref[...] += contrib
            cnt_ref[...] += cnt8

        @pl.when(i == nblk - 1)
        def _():
            cnt_f = jnp.maximum(cnt_ref[0:1, :], 1.0)
            pooled = pool_ref[...] / cnt_f.reshape(_G, 1)
            o = jnp.sum(pooled * wp_ref[...], axis=1, keepdims=True)
            o = o + bp_ref[0:1, 0:1]
            out_ref[...] = jnp.broadcast_to(o, (_G, 128))

    pool, cnt, out = pl.pallas_call(
        kern,
        grid=(nblk,),
        in_specs=[
            pl.BlockSpec((_BN, _D), lambda i: (i, 0)),
            pl.BlockSpec((1, 1, _BN), lambda i: (i, 0, 0)),
            pl.BlockSpec((1, _D), lambda i: (0, 0)),
            pl.BlockSpec((1, 1), lambda i: (0, 0)),
        ],
        out_specs=[
            pl.BlockSpec((_G, _D), lambda i: (0, 0)),
            pl.BlockSpec((8, _G), lambda i: (0, 0)),
            pl.BlockSpec((_G, 128), lambda i: (0, 0)),
        ],
        out_shape=[
            jax.ShapeDtypeStruct((_G, _D), jnp.float32),
            jax.ShapeDtypeStruct((8, _G), jnp.float32),
            jax.ShapeDtypeStruct((_G, 128), jnp.float32),
        ],
    )(h, batch2d, wp_row, bp)
    return out[:, 0:1]


def kernel(x, edge_index, batch, t0, W1_0, b1_0, gamma0, beta0, W2_0, b2_0,
           t1, W1_1, b1_1, gamma1, beta1, W2_1, b2_1,
           t2, W1_2, b1_2, gamma2, beta2, W2_2, b2_2, Wp, bp):
    srcdst3 = (edge_index[0] | (edge_index[1] << 16)).reshape(16, _NB, _EB)
    zrows = jnp.zeros((_RPT, 128), jnp.float32)
    h = x
    ts = [t0, t1, t2, jnp.array(1.0, jnp.float32)]
    xq = _tc_prep(x, ts[0].reshape(1, 1)).reshape(4 * _N, 64)
    layers = [
        (t0, W1_0, b1_0, gamma0, beta0, W2_0, b2_0),
        (t1, W1_1, b1_1, gamma1, beta1, W2_1, b2_1),
        (t2, W1_2, b1_2, gamma2, beta2, W2_2, b2_2),
    ]
    for li, (t, W1, b1, gamma, beta, W2, b2) in enumerate(layers):
        tvec = jnp.broadcast_to(t, (16,)).astype(jnp.float32)
        aggr = _sc_msgpass(xq, srcdst3, tvec, zrows)
        h1, stats = _tc_mlp1(aggr, h, W1, b1.reshape(1, _H))
        h, xq4 = _tc_mlp2(h1, stats, gamma.reshape(1, _H), beta.reshape(1, _H),
                          W2, b2.reshape(1, _D), h,
                          ts[li + 1].reshape(1, 1))
        xq = xq4.reshape(4 * _N, 64)
    return _tc_pool(h, batch.reshape(_N // _BN, 1, _BN), Wp.reshape(1, _D),
                    bp.reshape(1, 1))
